# baseline (speedup 1.0000x reference)
import functools

import jax
import jax.numpy as jnp
from jax import lax
from jax.experimental import pallas as pl
from jax.experimental.pallas import tpu as pltpu
from jax.experimental.pallas import tpu_sc as plsc

N_NODES = 10000
N_EDGES = 320000
D = 128
DH = D // 2
NC = 2
NS = 16
E_PER_TILE = N_EDGES // NS
G = 80
NCHUNK = E_PER_TILE // G
N_PAD = 10240
ROWS_PER_TILE = N_PAD // NS
ZR = 128


def _sc_partials(rows, cols, edge_vals, xs):
    mesh = plsc.VectorSubcoreMesh(core_axis_name="c", subcore_axis_name="s")

    @functools.partial(
        pl.kernel,
        mesh=mesh,
        compiler_params=pltpu.CompilerParams(use_tc_tiling_on_sc=False),
        out_type=jax.ShapeDtypeStruct((NC, N_PAD, DH), jnp.float32),
        scratch_types=[
            pltpu.VMEM((NCHUNK, G), jnp.int32),
            pltpu.VMEM((NCHUNK, G), jnp.int32),
            pltpu.VMEM((NCHUNK, G), jnp.float32),
            pltpu.VMEM((2, G, DH), jnp.float32),
            pltpu.VMEM((ZR, DH), jnp.float32),
            pltpu.VMEM_SHARED((N_PAD, DH), jnp.float32),
            pltpu.SemaphoreType.DMA((2,)),
            pltpu.SemaphoreType.DMA((2,)),
        ],
    )
    def body(rows_hbm, cols_hbm, ev_hbm, xs_hbm, out_hbm, rows_v, cols_v,
             vals_v, gbuf, zbuf, acc, gsem, ssem):
        cid = lax.axis_index("c")
        sid = lax.axis_index("s")

        pltpu.sync_copy(rows_hbm.at[sid], rows_v)
        pltpu.sync_copy(cols_hbm.at[sid], cols_v)
        pltpu.sync_copy(ev_hbm.at[sid], vals_v)

        zrow = jnp.zeros((16,), jnp.float32)

        def zinit(i, carry):
            for j in range(DH // 16):
                zbuf[i, pl.ds(j * 16, 16)] = zrow
            return carry

        lax.fori_loop(0, ZR, zinit, 0)
        for k in range(ROWS_PER_TILE // ZR):
            pltpu.sync_copy(
                zbuf, acc.at[pl.ds(sid * ROWS_PER_TILE + k * ZR, ZR)])
        plsc.subcore_barrier()

        def gather_cp(i, b):
            return pltpu.make_async_copy(
                xs_hbm.at[cid].at[cols_v.at[i]], gbuf.at[b], gsem.at[b])

        def scatter_cp(i, b):
            return pltpu.make_async_copy(
                gbuf.at[b], acc.at[rows_v.at[i]], ssem.at[b])

        def scale(i, b):
            def scale_g(g, c2):
                v16 = vals_v[i, pl.ds(g * 16, 16)]
                for l in range(16):
                    s = v16[l]
                    e = g * 16 + l
                    for j in range(DH // 16):
                        gbuf[b, e, pl.ds(j * 16, 16)] = (
                            gbuf[b, e, pl.ds(j * 16, 16)] * s)
                return c2

            lax.fori_loop(0, G // 16, scale_g, 0)

        gather_cp(0, 0).start()

        def step(i, carry):
            b = lax.rem(i, 2)
            ob = 1 - b
            gather_cp(i, b).wait()

            @pl.when(i >= 1)
            def _():
                scatter_cp(i - 1, ob).wait()

            @pl.when(i + 1 < NCHUNK)
            def _():
                gather_cp(i + 1, ob).start()

            scale(i, b)
            scatter_cp(i, b).start(add=True)
            return carry

        lax.fori_loop(0, NCHUNK, step, 0)
        scatter_cp(NCHUNK - 1, (NCHUNK - 1) % 2).wait()
        plsc.subcore_barrier()

        pltpu.sync_copy(
            acc.at[pl.ds(sid * ROWS_PER_TILE, ROWS_PER_TILE)],
            out_hbm.at[cid, pl.ds(sid * ROWS_PER_TILE, ROWS_PER_TILE)])

    return body(rows, cols, edge_vals, xs)


def _tc_concat(partials):
    def body(a_ref, b_ref, o_ref):
        o_ref[:, :DH] = a_ref[...]
        o_ref[:, DH:] = b_ref[...]

    return pl.pallas_call(
        body,
        grid=(10,),
        in_specs=[
            pl.BlockSpec((N_PAD // 10, DH), lambda i: (i, 0)),
            pl.BlockSpec((N_PAD // 10, DH), lambda i: (i, 0)),
        ],
        out_specs=pl.BlockSpec((N_PAD // 10, D), lambda i: (i, 0)),
        out_shape=jax.ShapeDtypeStruct((N_PAD, D), jnp.float32),
    )(partials[0], partials[1])


def kernel(edge_index, edge_vals, x):
    rows = edge_index[0].reshape(NS, NCHUNK, G)
    cols = edge_index[1].reshape(NS, NCHUNK, G)
    vals = edge_vals.reshape(NS, NCHUNK, G)
    xs = x.reshape(N_NODES, 2, DH).transpose(1, 0, 2)
    partials = _sc_partials(rows, cols, vals, xs)
    return _tc_concat(partials)[:N_NODES]
